# baseline (speedup 1.0000x reference)
import functools

import jax
import jax.numpy as jnp
from jax import lax
from jax.experimental import pallas as pl
from jax.experimental.pallas import tpu as pltpu
from jax.experimental.pallas import tpu_sc as plsc

B, L, V = 4096, 200, 1000000
VT = 1 << 20
VC = 1000064
N = B * L
NC, NS, LANES = 2, 16, 16
NW = NC * NS
PER_W = N // NW

PROD_PER_S = 62496
PROD_CHUNK = 3472
PROD_NCHUNK = 18
PROD_TAIL_OFF = NS * PROD_PER_S
PROD_TAIL = VC - PROD_TAIL_OFF


def _body(idx_hbm, tf_hbm, idf_hbm, out_hbm,
          idx_v, val_v, tf0, idf0, tf1, idf1, tl_tf, tl_idf, comb_sh,
          sem_idx, sem_g, sem_in0, sem_in1, sem_out0, sem_out1):
    c = lax.axis_index("c")
    s = lax.axis_index("s")
    wid = s * NC + c
    gbase = wid * PER_W

    cp_idx = pltpu.async_copy(idx_hbm.at[pl.ds(gbase, PER_W)], idx_v,
                              sem_idx)

    pbase = s * PROD_PER_S
    slots = ((tf0, idf0, sem_in0, sem_out0), (tf1, idf1, sem_in1, sem_out1))

    def mul_chunk(n_vec, tf_ref, idf_ref):
        def step(i, _):
            sl = pl.ds(i * LANES, LANES)
            tf_ref[sl] = tf_ref[sl] * idf_ref[sl]
            return 0
        lax.fori_loop(0, n_vec, step, 0, unroll=8)

    def start_in(k):
        tf_b, idf_b, s_in, _ = slots[k % 2]
        off = pbase + k * PROD_CHUNK
        a = pltpu.async_copy(tf_hbm.at[pl.ds(off, PROD_CHUNK)], tf_b, s_in)
        b = pltpu.async_copy(idf_hbm.at[pl.ds(off, PROD_CHUNK)], idf_b, s_in)
        return a, b

    pend_in = [None, None]
    pend_out = [None, None]
    pend_in[0] = start_in(0)
    for k in range(PROD_NCHUNK):
        tf_b, idf_b, _, s_out = slots[k % 2]
        if k + 1 < PROD_NCHUNK:
            nxt = (k + 1) % 2
            if pend_out[nxt] is not None:
                pend_out[nxt].wait()
                pend_out[nxt] = None
            pend_in[nxt] = start_in(k + 1)
        a, b = pend_in[k % 2]
        a.wait()
        b.wait()
        mul_chunk(PROD_CHUNK // LANES, tf_b, idf_b)
        off = pbase + k * PROD_CHUNK
        pend_out[k % 2] = pltpu.async_copy(
            tf_b, comb_sh.at[pl.ds(off, PROD_CHUNK)], s_out)
    for p in pend_out:
        if p is not None:
            p.wait()

    @pl.when(s == NS - 1)
    def _tail():
        pltpu.sync_copy(tf_hbm.at[pl.ds(PROD_TAIL_OFF, PROD_TAIL)], tl_tf)
        pltpu.sync_copy(idf_hbm.at[pl.ds(PROD_TAIL_OFF, PROD_TAIL)], tl_idf)
        mul_chunk(PROD_TAIL // LANES, tl_tf, tl_idf)
        pltpu.sync_copy(tl_tf, comb_sh.at[pl.ds(PROD_TAIL_OFF, PROD_TAIL)])

    plsc.subcore_barrier()

    cp_idx.wait()
    pltpu.async_copy(comb_sh.at[idx_v], val_v, sem_g).wait()
    pltpu.sync_copy(val_v, out_hbm.at[pl.ds(gbase, PER_W)])


@jax.jit
def _tfidf(idx_flat, tf_lin, idf_lin):
    mesh = plsc.VectorSubcoreMesh(core_axis_name="c", subcore_axis_name="s")
    fn = pl.kernel(
        _body,
        out_type=jax.ShapeDtypeStruct((N,), jnp.float32),
        mesh=mesh,
        scratch_types=[
            pltpu.VMEM((PER_W,), jnp.int32),
            pltpu.VMEM((PER_W,), jnp.float32),
            pltpu.VMEM((PROD_CHUNK,), jnp.float32),
            pltpu.VMEM((PROD_CHUNK,), jnp.float32),
            pltpu.VMEM((PROD_CHUNK,), jnp.float32),
            pltpu.VMEM((PROD_CHUNK,), jnp.float32),
            pltpu.VMEM((PROD_TAIL,), jnp.float32),
            pltpu.VMEM((PROD_TAIL,), jnp.float32),
            pltpu.VMEM_SHARED((VC,), jnp.float32),
            pltpu.SemaphoreType.DMA,
            pltpu.SemaphoreType.DMA,
            pltpu.SemaphoreType.DMA,
            pltpu.SemaphoreType.DMA,
            pltpu.SemaphoreType.DMA,
            pltpu.SemaphoreType.DMA,
        ],
    )
    return fn(idx_flat, tf_lin, idf_lin)


def kernel(indices, w_es, instance, tf_table, idf_table):
    idx_flat = indices.reshape(N).astype(jnp.int32)
    tf_lin = jnp.pad(tf_table, ((0, VT - V), (0, 0))).reshape(VT)
    idf_lin = jnp.pad(idf_table, ((0, VT - V), (0, 0))).reshape(VT)
    out = _tfidf(idx_flat, tf_lin, idf_lin)
    return out.reshape(B, L)


# speedup vs baseline: 1.0452x; 1.0452x over previous
import functools

import jax
import jax.numpy as jnp
from jax import lax
from jax.experimental import pallas as pl
from jax.experimental.pallas import tpu as pltpu
from jax.experimental.pallas import tpu_sc as plsc

B, L, V = 4096, 200, 1000000
VT = 1 << 20
VC = 1000064
N = B * L
NC, NS, LANES = 2, 16, 16
NW = NC * NS
PER_W = N // NW

PROD_PER_S = 62496
PROD_CHUNK = 10416
PROD_NCHUNK = 6
PROD_TAIL_OFF = NS * PROD_PER_S
PROD_TAIL = VC - PROD_TAIL_OFF


def _body(idx_hbm, tf_hbm, idf_hbm, out_hbm,
          idx_v, val_v, comb_sh, sem, sem_idx):
    c = lax.axis_index("c")
    s = lax.axis_index("s")
    wid = s * NC + c
    gbase = wid * PER_W

    cp_idx = pltpu.async_copy(idx_hbm.at[pl.ds(gbase, PER_W)], idx_v,
                              sem_idx)

    tf_c = val_v.at[pl.ds(0, PROD_CHUNK)]
    idf_c = val_v.at[pl.ds(12800, PROD_CHUNK)]
    pbase = s * PROD_PER_S

    def mul_chunk(n_vec, tf_ref, idf_ref):
        def step(i, _):
            sl = pl.ds(i * LANES, LANES)
            tf_ref[sl] = tf_ref[sl] * idf_ref[sl]
            return 0
        lax.fori_loop(0, n_vec, step, 0, unroll=8)

    for k in range(PROD_NCHUNK):
        off = pbase + k * PROD_CHUNK
        pltpu.sync_copy(tf_hbm.at[pl.ds(off, PROD_CHUNK)], tf_c)
        pltpu.sync_copy(idf_hbm.at[pl.ds(off, PROD_CHUNK)], idf_c)
        mul_chunk(PROD_CHUNK // LANES, tf_c, idf_c)
        pltpu.sync_copy(tf_c, comb_sh.at[pl.ds(off, PROD_CHUNK)])

    @pl.when(s == NS - 1)
    def _tail():
        tf_t = tf_c.at[pl.ds(0, PROD_TAIL)]
        idf_t = idf_c.at[pl.ds(0, PROD_TAIL)]
        pltpu.sync_copy(tf_hbm.at[pl.ds(PROD_TAIL_OFF, PROD_TAIL)], tf_t)
        pltpu.sync_copy(idf_hbm.at[pl.ds(PROD_TAIL_OFF, PROD_TAIL)], idf_t)
        mul_chunk(PROD_TAIL // LANES, tf_t, idf_t)
        pltpu.sync_copy(tf_t, comb_sh.at[pl.ds(PROD_TAIL_OFF, PROD_TAIL)])

    plsc.subcore_barrier()

    cp_idx.wait()
    pltpu.async_copy(comb_sh.at[idx_v], val_v, sem).wait()
    pltpu.sync_copy(val_v, out_hbm.at[pl.ds(gbase, PER_W)])


@jax.jit
def _tfidf(idx_flat, tf_lin, idf_lin):
    mesh = plsc.VectorSubcoreMesh(core_axis_name="c", subcore_axis_name="s")
    fn = pl.kernel(
        _body,
        out_type=jax.ShapeDtypeStruct((N,), jnp.float32),
        mesh=mesh,
        scratch_types=[
            pltpu.VMEM((PER_W,), jnp.int32),
            pltpu.VMEM((PER_W,), jnp.float32),
            pltpu.VMEM_SHARED((VC,), jnp.float32),
            pltpu.SemaphoreType.DMA,
            pltpu.SemaphoreType.DMA,
        ],
    )
    return fn(idx_flat, tf_lin, idf_lin)


def kernel(indices, w_es, instance, tf_table, idf_table):
    idx_flat = indices.reshape(N).astype(jnp.int32)
    tf_lin = jnp.pad(tf_table, ((0, VT - V), (0, 0))).reshape(VT)
    idf_lin = jnp.pad(idf_table, ((0, VT - V), (0, 0))).reshape(VT)
    out = _tfidf(idx_flat, tf_lin, idf_lin)
    return out.reshape(B, L)
